# baseline (speedup 1.0000x reference)
import functools

import jax
import jax.numpy as jnp
from jax import lax
from jax.experimental import pallas as pl
from jax.experimental.pallas import tpu as pltpu
from jax.experimental.pallas import tpu_sc as plsc

N_TOK = 4096
K_CB = 8192
D_LAT = 32

RB_ARGMAX = 256
RB_CDIST = 512

_SC_CORES = 2
_SC_SUBCORES = 16
_SC_WORKERS = _SC_CORES * _SC_SUBCORES
_TOK_PER_W = N_TOK // _SC_WORKERS


def _prep_body(cb_ref, cm_ref, cs_ref, x_ref, rep_ref, rn_ref, r2t_ref,
               xn_ref):
    rep = cm_ref[...] + cs_ref[...] * cb_ref[...]
    rep_ref[...] = rep
    r2col = jnp.sum(rep * rep, axis=1, keepdims=True)
    n = jnp.sqrt(r2col)
    rn_ref[...] = rep / jnp.maximum(n, 1e-12)
    ones = jnp.ones((1, D_LAT), jnp.float32)
    r2t_ref[...] = lax.dot_general(ones, rep * rep,
                                   (((1,), (1,)), ((), ())),
                                   precision=lax.Precision.HIGHEST,
                                   preferred_element_type=jnp.float32)
    x = x_ref[...]
    xn2 = jnp.sum(x * x, axis=1, keepdims=True)
    xn_ref[...] = x / jnp.maximum(jnp.sqrt(xn2), 1e-12)


def _prep(codebook, c_mean, c_std, x):
    return pl.pallas_call(
        _prep_body,
        out_shape=(
            jax.ShapeDtypeStruct((K_CB, D_LAT), jnp.float32),
            jax.ShapeDtypeStruct((K_CB, D_LAT), jnp.float32),
            jax.ShapeDtypeStruct((1, K_CB), jnp.float32),
            jax.ShapeDtypeStruct((N_TOK, D_LAT), jnp.float32),
        ),
    )(codebook, c_mean.reshape(1, D_LAT), c_std.reshape(1, D_LAT), x)


def _argmax_body(x_ref, g_ref, rep_ref, r2t_ref, idx_ref):
    x = x_ref[...]
    mm = lax.dot_general(x.astype(jnp.bfloat16),
                         rep_ref[...].astype(jnp.bfloat16),
                         (((1,), (1,)), ((), ())),
                         preferred_element_type=jnp.float32)
    x2 = jnp.sum(x * x, axis=1, keepdims=True)
    d = x2 - 2.0 * mm + r2t_ref[...]
    v = g_ref[...] + (-d)
    m = jnp.max(v, axis=1, keepdims=True)
    col = lax.broadcasted_iota(jnp.int32, v.shape, 1)
    idx = jnp.min(jnp.where(v == m, col, K_CB), axis=1, keepdims=True)
    idx_ref[0] = idx


def _argmax_cdist_body(x_ref, g_ref, repb_ref, rep_ref, r2t_ref,
                       idx_ref, sum_ref, min_ref):
    _argmax_body(x_ref, g_ref, rep_ref, r2t_ref, idx_ref)
    _cdist_body(repb_ref, rep_ref, r2t_ref, sum_ref, min_ref)


def _argmax_cdist(x, gumbel, rep, r2t):
    nblk = N_TOK // RB_ARGMAX
    assert K_CB // RB_CDIST == nblk
    scal = jax.ShapeDtypeStruct((1, 1), jnp.float32)
    smem = pl.BlockSpec((1, 1), lambda i: (0, 0), memory_space=pltpu.SMEM)
    idx3, sums, mins = pl.pallas_call(
        _argmax_cdist_body,
        grid=(nblk,),
        in_specs=[
            pl.BlockSpec((RB_ARGMAX, D_LAT), lambda i: (i, 0)),
            pl.BlockSpec((RB_ARGMAX, K_CB), lambda i: (i, 0)),
            pl.BlockSpec((RB_CDIST, D_LAT), lambda i: (i, 0)),
            pl.BlockSpec((K_CB, D_LAT), lambda i: (0, 0)),
            pl.BlockSpec((1, K_CB), lambda i: (0, 0)),
        ],
        out_specs=(
            pl.BlockSpec((1, RB_ARGMAX, 1), lambda i: (i, 0, 0)),
            smem, smem,
        ),
        out_shape=(
            jax.ShapeDtypeStruct((nblk, RB_ARGMAX, 1), jnp.int32),
            scal, scal,
        ),
    )(x, gumbel, rep, rep, r2t)
    return idx3.reshape(N_TOK), sums, mins


def _cdist_body(repb_ref, rep_ref, r2t_ref, sum_ref, min_ref):
    j = pl.program_id(0)
    repb = repb_ref[...]
    mm = lax.dot_general(repb.astype(jnp.bfloat16),
                         rep_ref[...].astype(jnp.bfloat16),
                         (((1,), (1,)), ((), ())),
                         preferred_element_type=jnp.float32)
    r2b = jnp.sum(repb * repb, axis=1, keepdims=True)
    sq = jnp.maximum(r2b + r2t_ref[...] - 2.0 * mm, 0.0)
    dd = jnp.sqrt(sq)
    row = j * RB_CDIST + lax.broadcasted_iota(jnp.int32, dd.shape, 0)
    col = lax.broadcasted_iota(jnp.int32, dd.shape, 1)
    diag = row == col
    s = jnp.sum(jnp.where(diag, 0.0, dd))
    mn = jnp.min(jnp.where(diag, jnp.inf, dd))

    @pl.when(j == 0)
    def _():
        sum_ref[0, 0] = s
        min_ref[0, 0] = mn

    @pl.when(j != 0)
    def _():
        sum_ref[0, 0] = sum_ref[0, 0] + s
        min_ref[0, 0] = jnp.minimum(min_ref[0, 0], mn)


def _cdist_stats(rep, r2t):
    nblk = K_CB // RB_CDIST
    return pl.pallas_call(
        _cdist_body,
        grid=(nblk,),
        in_specs=[
            pl.BlockSpec((RB_CDIST, D_LAT), lambda j: (j, 0)),
            pl.BlockSpec((K_CB, D_LAT), lambda j: (0, 0)),
            pl.BlockSpec((1, K_CB), lambda j: (0, 0)),
        ],
        out_specs=(
            pl.BlockSpec((1, 1), lambda j: (0, 0), memory_space=pltpu.SMEM),
            pl.BlockSpec((1, 1), lambda j: (0, 0), memory_space=pltpu.SMEM),
        ),
        out_shape=(
            jax.ShapeDtypeStruct((1, 1), jnp.float32),
            jax.ShapeDtypeStruct((1, 1), jnp.float32),
        ),
    )(rep, rep, r2t)


def _sc_body(rep_hbm, rn_hbm, idx_hbm, q_hbm, selrn_hbm, counts_hbm,
             idx_v, rows_v, ones_v, zblk_v, shared, sem):
    c = lax.axis_index("c")
    s = lax.axis_index("s")
    wid = s * _SC_CORES + c
    base = wid * _TOK_PER_W

    zrows = K_CB // _SC_SUBCORES

    def zbody(i, _):
        zblk_v[i, :] = jnp.zeros((16,), jnp.float32)
        return _

    lax.fori_loop(0, zrows, zbody, None)
    pltpu.sync_copy(zblk_v, shared.at[pl.ds(s * zrows, zrows)])

    def obody(i, _):
        ones_v[i, :] = jnp.ones((16,), jnp.float32)
        return _

    lax.fori_loop(0, _TOK_PER_W, obody, None)

    pltpu.sync_copy(idx_hbm.at[pl.ds(base, _TOK_PER_W)], idx_v)
    pltpu.async_copy(rep_hbm.at[idx_v], rows_v, sem).wait()
    pltpu.sync_copy(rows_v, q_hbm.at[pl.ds(base, _TOK_PER_W)])
    pltpu.async_copy(rn_hbm.at[idx_v], rows_v, sem).wait()
    pltpu.sync_copy(rows_v, selrn_hbm.at[pl.ds(base, _TOK_PER_W)])

    plsc.subcore_barrier()
    pltpu.async_copy(ones_v, shared.at[idx_v], sem, add=True).wait()
    plsc.subcore_barrier()

    @pl.when(s == 0)
    def _():
        pltpu.sync_copy(shared, counts_hbm.at[c])


@functools.partial(jax.jit, static_argnums=())
def _sc_gather_hist(rep, rn, idx):
    mesh = plsc.VectorSubcoreMesh(core_axis_name="c", subcore_axis_name="s")
    f = pl.kernel(
        _sc_body,
        out_type=(
            jax.ShapeDtypeStruct((N_TOK, D_LAT), jnp.float32),
            jax.ShapeDtypeStruct((N_TOK, D_LAT), jnp.float32),
            jax.ShapeDtypeStruct((_SC_CORES, K_CB, 16), jnp.float32),
        ),
        mesh=mesh,
        scratch_types=[
            pltpu.VMEM((_TOK_PER_W,), jnp.int32),
            pltpu.VMEM((_TOK_PER_W, D_LAT), jnp.float32),
            pltpu.VMEM((_TOK_PER_W, 16), jnp.float32),
            pltpu.VMEM((K_CB // _SC_SUBCORES, 16), jnp.float32),
            pltpu.VMEM_SHARED((K_CB, 16), jnp.float32),
            pltpu.SemaphoreType.DMA,
        ],
        compiler_params=pltpu.CompilerParams(use_tc_tiling_on_sc=False),
    )
    return f(rep, rn, idx)


def _final_body(x_ref, q_ref, xn_ref, selrn_ref, counts_ref, sum_ref,
                min_ref, qf_ref, commit_ref, cbl_ref, perp_ref, selcos_ref,
                avg_ref, minv_ref, ggap_ref):
    x = x_ref[...]
    q = q_ref[...].astype(jnp.bfloat16).astype(jnp.float32)
    diff = x - q
    mse = jnp.sum(diff * diff) / (N_TOK * D_LAT)
    commit_ref[0, 0] = 0.25 * mse
    cbl_ref[0, 0] = mse
    qf = x + (q - x)
    qf_ref[...] = qf
    gd = x - qf
    ggap_ref[0, 0] = jnp.sqrt(jnp.sum(gd * gd))
    xnb = xn_ref[...].astype(jnp.bfloat16).astype(jnp.float32)
    srb = selrn_ref[...].astype(jnp.bfloat16).astype(jnp.float32)
    selcos_ref[0, 0] = jnp.sum(xnb * srb) / N_TOK
    counts = counts_ref[0, :, 0:1] + counts_ref[1, :, 0:1]
    p = counts / N_TOK
    perp_ref[0, 0] = jnp.exp(-jnp.sum(p * jnp.log(p + 1e-10)))
    avg_ref[0, 0] = sum_ref[0, 0] / (K_CB * (K_CB - 1))
    minv_ref[0, 0] = min_ref[0, 0]


def _finalize(x, q, xn, selrn, counts, sums, mins):
    scal = jax.ShapeDtypeStruct((1, 1), jnp.float32)
    vmem = pl.BlockSpec(memory_space=pltpu.VMEM)
    smem = pl.BlockSpec(memory_space=pltpu.SMEM)
    return pl.pallas_call(
        _final_body,
        in_specs=[vmem, vmem, vmem, vmem, vmem, smem, smem],
        out_specs=(vmem, smem, smem, smem, smem, smem, smem, smem),
        out_shape=(
            jax.ShapeDtypeStruct((N_TOK, D_LAT), jnp.float32),
            scal, scal, scal, scal, scal, scal, scal,
        ),
    )(x, q, xn, selrn, counts, sums, mins)


def kernel(latent, codebook, c_mean, c_std):
    B, S, D = latent.shape
    x = latent.reshape(-1, D)
    rep, rn, r2t, xn = _prep(codebook, c_mean, c_std, x)
    skey = jax.random.fold_in(jax.random.key(0), 123)
    gumbel = jax.random.gumbel(skey, (N_TOK, K_CB), jnp.float32)
    idx, sums, mins = _argmax_cdist(x, gumbel, rep, r2t)
    q, selrn, counts = _sc_gather_hist(rep, rn, idx)
    (qf, commit, cbl, perp, selcos, avg, minv, ggap) = _finalize(
        x, q, xn, selrn, counts, sums, mins)
    return (qf.reshape(B, S, D), idx, commit[0, 0], cbl[0, 0], perp[0, 0],
            selcos[0, 0], avg[0, 0], minv[0, 0], ggap[0, 0])


# speedup vs baseline: 1.0167x; 1.0167x over previous
import functools

import jax
import jax.numpy as jnp
from jax import lax
from jax.experimental import pallas as pl
from jax.experimental.pallas import tpu as pltpu
from jax.experimental.pallas import tpu_sc as plsc

N_TOK = 4096
K_CB = 8192
D_LAT = 32

RB_ARGMAX = 256
RB_CDIST = 512

_SC_CORES = 2
_SC_SUBCORES = 16
_SC_WORKERS = _SC_CORES * _SC_SUBCORES
_TOK_PER_W = N_TOK // _SC_WORKERS


def _prep_body(cb_ref, cm_ref, cs_ref, x_ref, rep_ref, rn_ref, r2t_ref,
               xn_ref):
    rep = cm_ref[...] + cs_ref[...] * cb_ref[...]
    rep_ref[...] = rep
    r2col = jnp.sum(rep * rep, axis=1, keepdims=True)
    n = jnp.sqrt(r2col)
    rn_ref[...] = rep / jnp.maximum(n, 1e-12)
    ones = jnp.ones((1, D_LAT), jnp.float32)
    r2t_ref[...] = lax.dot_general(ones, rep * rep,
                                   (((1,), (1,)), ((), ())),
                                   precision=lax.Precision.HIGHEST,
                                   preferred_element_type=jnp.float32)
    x = x_ref[...]
    xn2 = jnp.sum(x * x, axis=1, keepdims=True)
    xn_ref[...] = x / jnp.maximum(jnp.sqrt(xn2), 1e-12)


def _prep(codebook, c_mean, c_std, x):
    return pl.pallas_call(
        _prep_body,
        out_shape=(
            jax.ShapeDtypeStruct((K_CB, D_LAT), jnp.float32),
            jax.ShapeDtypeStruct((K_CB, D_LAT), jnp.float32),
            jax.ShapeDtypeStruct((1, K_CB), jnp.float32),
            jax.ShapeDtypeStruct((N_TOK, D_LAT), jnp.float32),
        ),
    )(codebook, c_mean.reshape(1, D_LAT), c_std.reshape(1, D_LAT), x)


def _argmax_body(x_ref, g_ref, rep_ref, r2t_ref, idx_ref):
    x = x_ref[...]
    mm = lax.dot_general(x.astype(jnp.bfloat16),
                         rep_ref[...].astype(jnp.bfloat16),
                         (((1,), (1,)), ((), ())),
                         preferred_element_type=jnp.float32)
    x2 = jnp.sum(x * x, axis=1, keepdims=True)
    d = x2 - 2.0 * mm + r2t_ref[...]
    v = g_ref[...] + (-d)
    m = jnp.max(v, axis=1, keepdims=True)
    col = lax.broadcasted_iota(jnp.int32, v.shape, 1)
    idx = jnp.min(jnp.where(v == m, col, K_CB), axis=1, keepdims=True)
    idx_ref[0] = idx


def _argmax(x, gumbel, rep, r2t):
    nblk = N_TOK // RB_ARGMAX
    out = pl.pallas_call(
        _argmax_body,
        grid=(nblk,),
        in_specs=[
            pl.BlockSpec((RB_ARGMAX, D_LAT), lambda i: (i, 0)),
            pl.BlockSpec((RB_ARGMAX, K_CB), lambda i: (i, 0)),
            pl.BlockSpec((K_CB, D_LAT), lambda i: (0, 0)),
            pl.BlockSpec((1, K_CB), lambda i: (0, 0)),
        ],
        out_specs=pl.BlockSpec((1, RB_ARGMAX, 1), lambda i: (i, 0, 0)),
        out_shape=jax.ShapeDtypeStruct((nblk, RB_ARGMAX, 1), jnp.int32),
    )(x, gumbel, rep, r2t)
    return out.reshape(N_TOK)


def _cdist_body(repb_ref, rep_ref, r2t_ref, sum_ref, min_ref):
    j = pl.program_id(0)
    repb = repb_ref[...]
    mm = lax.dot_general(repb.astype(jnp.bfloat16),
                         rep_ref[...].astype(jnp.bfloat16),
                         (((1,), (1,)), ((), ())),
                         preferred_element_type=jnp.float32)
    r2b = jnp.sum(repb * repb, axis=1, keepdims=True)
    sq = jnp.maximum(r2b + r2t_ref[...] - 2.0 * mm, 0.0)
    row = j * RB_CDIST + lax.broadcasted_iota(jnp.int32, sq.shape, 0)
    col = lax.broadcasted_iota(jnp.int32, sq.shape, 1)
    dd = jnp.sqrt(jnp.where(row == col, 1.0, sq))
    s = jnp.sum(dd) - jnp.float32(RB_CDIST)
    mn = jnp.min(dd)

    @pl.when(j == 0)
    def _():
        sum_ref[0, 0] = s
        min_ref[0, 0] = mn

    @pl.when(j != 0)
    def _():
        sum_ref[0, 0] = sum_ref[0, 0] + s
        min_ref[0, 0] = jnp.minimum(min_ref[0, 0], mn)


def _cdist_stats(rep, r2t):
    nblk = K_CB // RB_CDIST
    return pl.pallas_call(
        _cdist_body,
        grid=(nblk,),
        in_specs=[
            pl.BlockSpec((RB_CDIST, D_LAT), lambda j: (j, 0)),
            pl.BlockSpec((K_CB, D_LAT), lambda j: (0, 0)),
            pl.BlockSpec((1, K_CB), lambda j: (0, 0)),
        ],
        out_specs=(
            pl.BlockSpec((1, 1), lambda j: (0, 0), memory_space=pltpu.SMEM),
            pl.BlockSpec((1, 1), lambda j: (0, 0), memory_space=pltpu.SMEM),
        ),
        out_shape=(
            jax.ShapeDtypeStruct((1, 1), jnp.float32),
            jax.ShapeDtypeStruct((1, 1), jnp.float32),
        ),
    )(rep, rep, r2t)


def _sc_body(rep_hbm, rn_hbm, idx_hbm, q_hbm, selrn_hbm, counts_hbm,
             idx_v, rows_v, ones_v, zblk_v, shared, sem):
    c = lax.axis_index("c")
    s = lax.axis_index("s")
    wid = s * _SC_CORES + c
    base = wid * _TOK_PER_W

    zrows = K_CB // _SC_SUBCORES

    def zbody(i, _):
        zblk_v[i, :] = jnp.zeros((16,), jnp.float32)
        return _

    lax.fori_loop(0, zrows, zbody, None)
    pltpu.sync_copy(zblk_v, shared.at[pl.ds(s * zrows, zrows)])

    def obody(i, _):
        ones_v[i, :] = jnp.ones((16,), jnp.float32)
        return _

    lax.fori_loop(0, _TOK_PER_W, obody, None)

    pltpu.sync_copy(idx_hbm.at[pl.ds(base, _TOK_PER_W)], idx_v)
    pltpu.async_copy(rep_hbm.at[idx_v], rows_v, sem).wait()
    pltpu.sync_copy(rows_v, q_hbm.at[pl.ds(base, _TOK_PER_W)])
    pltpu.async_copy(rn_hbm.at[idx_v], rows_v, sem).wait()
    pltpu.sync_copy(rows_v, selrn_hbm.at[pl.ds(base, _TOK_PER_W)])

    plsc.subcore_barrier()
    pltpu.async_copy(ones_v, shared.at[idx_v], sem, add=True).wait()
    plsc.subcore_barrier()

    @pl.when(s == 0)
    def _():
        pltpu.sync_copy(shared, counts_hbm.at[c])


@functools.partial(jax.jit, static_argnums=())
def _sc_gather_hist(rep, rn, idx):
    mesh = plsc.VectorSubcoreMesh(core_axis_name="c", subcore_axis_name="s")
    f = pl.kernel(
        _sc_body,
        out_type=(
            jax.ShapeDtypeStruct((N_TOK, D_LAT), jnp.float32),
            jax.ShapeDtypeStruct((N_TOK, D_LAT), jnp.float32),
            jax.ShapeDtypeStruct((_SC_CORES, K_CB, 16), jnp.float32),
        ),
        mesh=mesh,
        scratch_types=[
            pltpu.VMEM((_TOK_PER_W,), jnp.int32),
            pltpu.VMEM((_TOK_PER_W, D_LAT), jnp.float32),
            pltpu.VMEM((_TOK_PER_W, 16), jnp.float32),
            pltpu.VMEM((K_CB // _SC_SUBCORES, 16), jnp.float32),
            pltpu.VMEM_SHARED((K_CB, 16), jnp.float32),
            pltpu.SemaphoreType.DMA,
        ],
        compiler_params=pltpu.CompilerParams(use_tc_tiling_on_sc=False),
    )
    return f(rep, rn, idx)


def _final_body(x_ref, q_ref, xn_ref, selrn_ref, counts_ref, sum_ref,
                min_ref, qf_ref, commit_ref, cbl_ref, perp_ref, selcos_ref,
                avg_ref, minv_ref, ggap_ref):
    x = x_ref[...]
    q = q_ref[...].astype(jnp.bfloat16).astype(jnp.float32)
    diff = x - q
    mse = jnp.sum(diff * diff) / (N_TOK * D_LAT)
    commit_ref[0, 0] = 0.25 * mse
    cbl_ref[0, 0] = mse
    qf = x + (q - x)
    qf_ref[...] = qf
    gd = x - qf
    ggap_ref[0, 0] = jnp.sqrt(jnp.sum(gd * gd))
    xnb = xn_ref[...].astype(jnp.bfloat16).astype(jnp.float32)
    srb = selrn_ref[...].astype(jnp.bfloat16).astype(jnp.float32)
    selcos_ref[0, 0] = jnp.sum(xnb * srb) / N_TOK
    counts = counts_ref[0, :, 0:1] + counts_ref[1, :, 0:1]
    p = counts / N_TOK
    perp_ref[0, 0] = jnp.exp(-jnp.sum(p * jnp.log(p + 1e-10)))
    avg_ref[0, 0] = sum_ref[0, 0] / (K_CB * (K_CB - 1))
    minv_ref[0, 0] = min_ref[0, 0]


def _finalize(x, q, xn, selrn, counts, sums, mins):
    scal = jax.ShapeDtypeStruct((1, 1), jnp.float32)
    vmem = pl.BlockSpec(memory_space=pltpu.VMEM)
    smem = pl.BlockSpec(memory_space=pltpu.SMEM)
    return pl.pallas_call(
        _final_body,
        in_specs=[vmem, vmem, vmem, vmem, vmem, smem, smem],
        out_specs=(vmem, smem, smem, smem, smem, smem, smem, smem),
        out_shape=(
            jax.ShapeDtypeStruct((N_TOK, D_LAT), jnp.float32),
            scal, scal, scal, scal, scal, scal, scal,
        ),
    )(x, q, xn, selrn, counts, sums, mins)


def kernel(latent, codebook, c_mean, c_std):
    B, S, D = latent.shape
    x = latent.reshape(-1, D)
    rep, rn, r2t, xn = _prep(codebook, c_mean, c_std, x)
    skey = jax.random.fold_in(jax.random.key(0), 123)
    gumbel = jax.random.gumbel(skey, (N_TOK, K_CB), jnp.float32)
    idx = _argmax(x, gumbel, rep, r2t)
    sums, mins = _cdist_stats(rep, r2t)
    q, selrn, counts = _sc_gather_hist(rep, rn, idx)
    (qf, commit, cbl, perp, selcos, avg, minv, ggap) = _finalize(
        x, q, xn, selrn, counts, sums, mins)
    return (qf.reshape(B, S, D), idx, commit[0, 0], cbl[0, 0], perp[0, 0],
            selcos[0, 0], avg[0, 0], minv[0, 0], ggap[0, 0])
